# baseline (speedup 1.0000x reference)
import functools

import jax
import jax.numpy as jnp
from jax import lax
from jax.experimental import pallas as pl
from jax.experimental.pallas import tpu as pltpu
from jax.experimental.pallas import tpu_sc as plsc

N = 50000
E = 800000
D = 128
Hd = 64
B = 1024

EPAD = 802816
FACC = 50048
FSTRIPE = FACC // 16
HALF = 25000
ACC = 25088
ASTRIPE = ACC // 16
CH32 = EPAD // 32 // 128
CH16 = EPAD // 16 // 128
BLK16 = EPAD // 16 // 1024
BLK16G = EPAD // 16 // 512

_mesh = plsc.VectorSubcoreMesh(core_axis_name="c", subcore_axis_name="s")
f32 = jnp.float32
i32 = jnp.int32


def _lrelu(x):
    return jnp.maximum(x, 0.2 * x)


def _ln(h, g, b):
    m = jnp.mean(h, axis=-1, keepdims=True)
    v = jnp.mean((h - m) ** 2, axis=-1, keepdims=True)
    return (h - m) * lax.rsqrt(v + 1e-5) * g + b


@functools.partial(
    pl.kernel, mesh=_mesh,
    compiler_params=pltpu.CompilerParams(use_tc_tiling_on_sc=False),
    out_type=jax.ShapeDtypeStruct((2, FACC, 16), f32),
    scratch_types=[
        pltpu.VMEM((128,), i32),
        pltpu.VMEM((128, 16), f32),
        pltpu.VMEM_SHARED((FACC, 16), f32),
    ],
)
def _sc_degree(c_hbm, wrows_hbm, z16_hbm, out_hbm, cv, buf, acc):
    sc = lax.axis_index("c")
    s = lax.axis_index("s")
    wid = s * 2 + sc
    pltpu.sync_copy(z16_hbm.at[pl.ds(s * FSTRIPE, FSTRIPE)],
                    acc.at[pl.ds(s * FSTRIPE, FSTRIPE)])
    plsc.subcore_barrier()

    def chunk(g, _):
        off = (wid * CH32 + g) * 128
        pltpu.sync_copy(c_hbm.at[pl.ds(off, 128)], cv)
        pltpu.sync_copy(wrows_hbm.at[pl.ds(off, 128)], buf)
        pltpu.sync_copy(buf, acc.at[cv], add=True)
        return _

    lax.fori_loop(0, CH32, chunk, None)
    plsc.subcore_barrier()
    pltpu.sync_copy(acc.at[pl.ds(s * FSTRIPE, FSTRIPE)],
                    out_hbm.at[sc, pl.ds(s * FSTRIPE, FSTRIPE)])


@functools.partial(
    pl.kernel, mesh=_mesh,
    compiler_params=pltpu.CompilerParams(use_tc_tiling_on_sc=False),
    out_type=jax.ShapeDtypeStruct((2, ACC, 64), f32),
    scratch_types=[
        pltpu.VMEM((1024,), i32),
        pltpu.VMEM((1024,), i32),
        pltpu.VMEM((128,), i32),
        pltpu.VMEM((1040,), f32),
        pltpu.VMEM((128, 64), f32),
        pltpu.VMEM((128, 64), f32),
        pltpu.VMEM_SHARED((ACC, 64), f32),
        pltpu.SemaphoreType.DMA,
        pltpu.SemaphoreType.DMA,
    ],
)
def _sc_gcn_agg(table_hbm, r_hbm, c_hbm, w_hbm, z64_hbm, out_hbm,
                rv, cv, cl, wv, rows0, rows1, acc, sem0, sem1):
    sc = lax.axis_index("c")
    s = lax.axis_index("s")
    base = sc * HALF
    pltpu.sync_copy(z64_hbm.at[pl.ds(s * ASTRIPE, ASTRIPE)],
                    acc.at[pl.ds(s * ASTRIPE, ASTRIPE)])
    plsc.subcore_barrier()

    bufs = None

    def block(g, _):
        off = (s * BLK16 + g) * 1024
        pltpu.sync_copy(r_hbm.at[pl.ds(off, 1024)], rv)
        pltpu.sync_copy(c_hbm.at[pl.ds(off, 1024)], cv)
        pltpu.sync_copy(w_hbm.at[pl.ds(off, 1024)], wv.at[pl.ds(0, 1024)])
        bufs = (rows0, rows1)
        sems = (sem0, sem1)

        def fire(sub):
            return pltpu.async_copy(
                table_hbm.at[rv.at[pl.ds(sub * 128, 128)]],
                bufs[sub % 2], sems[sub % 2])

        hs = [None] * 8
        hs[0] = fire(0)
        for sub in range(8):
            if sub + 1 < 8:
                hs[sub + 1] = fire(sub + 1)
            hs[sub].wait()
            rows = bufs[sub % 2]
            sb = sub * 128
            for j in range(8):
                lv = cv[pl.ds(sb + j * 16, 16)] - base
                ok = (lv >= 0) & (lv < HALF)
                cl[pl.ds(j * 16, 16)] = jnp.where(ok, lv, HALF)

            def edge(e, __, rows=rows, sb=sb):
                w = wv[pl.ds(sb + e, 16)][0]
                for k in range(4):
                    sl = rows[e, pl.ds(k * 16, 16)]
                    rows[e, pl.ds(k * 16, 16)] = sl * w
                return __

            lax.fori_loop(0, 128, edge, None, unroll=4)
            pltpu.sync_copy(rows, acc.at[cl], add=True)
        return _

    lax.fori_loop(0, BLK16, block, None)
    plsc.subcore_barrier()
    pltpu.sync_copy(acc.at[pl.ds(s * ASTRIPE, ASTRIPE)],
                    out_hbm.at[sc, pl.ds(s * ASTRIPE, ASTRIPE)])


@functools.partial(
    pl.kernel, mesh=_mesh,
    compiler_params=pltpu.CompilerParams(use_tc_tiling_on_sc=False),
    out_type=jax.ShapeDtypeStruct((2, FACC, 16), f32),
    scratch_types=[
        pltpu.VMEM((128,), i32),
        pltpu.VMEM((128,), i32),
        pltpu.VMEM((128, 16), f32),
        pltpu.VMEM((128, 16), f32),
        pltpu.VMEM((128, 16), f32),
        pltpu.VMEM_SHARED((FACC, 16), f32),
        pltpu.SemaphoreType.DMA,
    ],
)
def _sc_gat_denom(ts_hbm, td_hbm, r_hbm, c_hbm, z16_hbm, out_hbm,
                  rv, cv, srows, drows, pbuf, acc, sem):
    sc = lax.axis_index("c")
    s = lax.axis_index("s")
    wid = s * 2 + sc
    pltpu.sync_copy(z16_hbm.at[pl.ds(s * FSTRIPE, FSTRIPE)],
                    acc.at[pl.ds(s * FSTRIPE, FSTRIPE)])
    plsc.subcore_barrier()

    def chunk(g, _):
        off = (wid * CH32 + g) * 128
        pltpu.sync_copy(r_hbm.at[pl.ds(off, 128)], rv)
        pltpu.sync_copy(c_hbm.at[pl.ds(off, 128)], cv)
        pltpu.async_copy(ts_hbm.at[rv], srows, sem).wait()
        pltpu.async_copy(td_hbm.at[cv], drows, sem).wait()

        def edge(e, __):
            ev = srows[e, pl.ds(0, 16)] + drows[e, pl.ds(0, 16)]
            pbuf[e, pl.ds(0, 16)] = jnp.exp(_lrelu(ev))
            return __

        lax.fori_loop(0, 128, edge, None, unroll=4)
        pltpu.sync_copy(pbuf, acc.at[cv], add=True)
        return _

    lax.fori_loop(0, CH32, chunk, None)
    plsc.subcore_barrier()
    pltpu.sync_copy(acc.at[pl.ds(s * FSTRIPE, FSTRIPE)],
                    out_hbm.at[sc, pl.ds(s * FSTRIPE, FSTRIPE)])


@functools.partial(
    pl.kernel, mesh=_mesh,
    compiler_params=pltpu.CompilerParams(use_tc_tiling_on_sc=False),
    out_type=jax.ShapeDtypeStruct((2, ACC, 64), f32),
    scratch_types=[
        pltpu.VMEM((512,), i32),
        pltpu.VMEM((512,), i32),
        pltpu.VMEM((64,), i32),
        pltpu.VMEM((64, 144), f32),
        pltpu.VMEM((64, 144), f32),
        pltpu.VMEM((64, 16), f32),
        pltpu.VMEM((64, 16), f32),
        pltpu.VMEM((64, 64), f32),
        pltpu.VMEM_SHARED((ACC, 64), f32),
        pltpu.SemaphoreType.DMA,
        pltpu.SemaphoreType.DMA,
        pltpu.SemaphoreType.DMA,
        pltpu.SemaphoreType.DMA,
    ],
)
def _sc_gat_agg(tg_hbm, dg_hbm, r_hbm, c_hbm, z64_hbm, out_hbm,
                rv, cv, cl, trows0, trows1, drows0, drows1, msg, acc,
                semt0, semt1, semd0, semd1):
    sc = lax.axis_index("c")
    s = lax.axis_index("s")
    base = sc * HALF
    pltpu.sync_copy(z64_hbm.at[pl.ds(s * ASTRIPE, ASTRIPE)],
                    acc.at[pl.ds(s * ASTRIPE, ASTRIPE)])
    plsc.subcore_barrier()

    def block(g, _):
        off = (s * BLK16G + g) * 512
        pltpu.sync_copy(r_hbm.at[pl.ds(off, 512)], rv)
        pltpu.sync_copy(c_hbm.at[pl.ds(off, 512)], cv)
        tbufs = (trows0, trows1)
        dbufs = (drows0, drows1)
        tsems = (semt0, semt1)
        dsems = (semd0, semd1)

        def fire(sub):
            ht = pltpu.async_copy(tg_hbm.at[rv.at[pl.ds(sub * 64, 64)]],
                                  tbufs[sub % 2], tsems[sub % 2])
            hd = pltpu.async_copy(dg_hbm.at[cv.at[pl.ds(sub * 64, 64)]],
                                  dbufs[sub % 2], dsems[sub % 2])
            return ht, hd

        hs = [None] * 8
        hs[0] = fire(0)
        for sub in range(8):
            if sub + 1 < 8:
                hs[sub + 1] = fire(sub + 1)
            hs[sub][0].wait()
            hs[sub][1].wait()
            trows = tbufs[sub % 2]
            drows = dbufs[sub % 2]
            sb = sub * 64
            for j in range(4):
                lv = cv[pl.ds(sb + j * 16, 16)] - base
                ok = (lv >= 0) & (lv < HALF)
                cl[pl.ds(j * 16, 16)] = jnp.where(ok, lv, HALF)

            def edge(e, __, trows=trows, drows=drows):
                dv = drows[e, pl.ds(0, 16)]
                ev = trows[e, pl.ds(128, 16)] + dv
                p = jnp.exp(_lrelu(ev))
                w0 = p[0] * dv[2]
                w1 = p[1] * dv[3]
                for k in range(4):
                    h0 = trows[e, pl.ds(k * 16, 16)]
                    h1 = trows[e, pl.ds(64 + k * 16, 16)]
                    msg[e, pl.ds(k * 16, 16)] = h0 * w0 + h1 * w1
                return __

            lax.fori_loop(0, 64, edge, None, unroll=4)
            pltpu.sync_copy(msg, acc.at[cl], add=True)
        return _

    lax.fori_loop(0, BLK16G, block, None)
    plsc.subcore_barrier()
    pltpu.sync_copy(acc.at[pl.ds(s * ASTRIPE, ASTRIPE)],
                    out_hbm.at[sc, pl.ds(s * ASTRIPE, ASTRIPE)])


@functools.partial(
    pl.kernel, mesh=_mesh,
    compiler_params=pltpu.CompilerParams(use_tc_tiling_on_sc=False),
    out_type=(jax.ShapeDtypeStruct((B, 64), f32),
              jax.ShapeDtypeStruct((B, 64), f32)),
    scratch_types=[
        pltpu.VMEM((32,), i32),
        pltpu.VMEM((32, 64), f32),
        pltpu.SemaphoreType.DMA,
    ],
)
def _sc_pair_gather(xf_hbm, src_hbm, dst_hbm, se_hbm, de_hbm, idx, rows, sem):
    sc = lax.axis_index("c")
    s = lax.axis_index("s")
    wid = s * 2 + sc
    bp = B // 32
    pltpu.sync_copy(src_hbm.at[pl.ds(wid * bp, bp)], idx)
    pltpu.async_copy(xf_hbm.at[idx], rows, sem).wait()
    pltpu.sync_copy(rows, se_hbm.at[pl.ds(wid * bp, bp)])
    pltpu.sync_copy(dst_hbm.at[pl.ds(wid * bp, bp)], idx)
    pltpu.async_copy(xf_hbm.at[idx], rows, sem).wait()
    pltpu.sync_copy(rows, de_hbm.at[pl.ds(wid * bp, bp)])


BM = 256
GRID = pl.cdiv(N, BM)


def _row_spec(w):
    return pl.BlockSpec((BM, w), lambda i: (i, 0))


def _full_spec(a, b):
    return pl.BlockSpec((a, b), lambda i: (0, 0))


def _tc_encode(x_ref, w1_ref, b1_ref, w2_ref, b2_ref, h_ref):
    t = jnp.maximum(x_ref[...] @ w1_ref[...] + b1_ref[...], 0.0)
    h_ref[...] = jnp.maximum(t @ w2_ref[...] + b2_ref[...], 0.0)


def _tc_deg_scale(h_ref, d0_ref, d1_ref, wg_ref, a1_ref, s1_ref, dis_ref):
    deg = 1.0 + d0_ref[...][:, 0:1] + d1_ref[...][:, 0:1]
    dis = lax.rsqrt(deg)
    hw = h_ref[...] @ wg_ref[...]
    a1_ref[...] = hw * dis
    s1_ref[...] = hw * (dis * dis)
    dis_ref[...] = dis


def _tc_gat_prep(agg_ref, s1_ref, dis_ref, b1_ref, lng_ref, lnb_ref,
                 wgat_ref, as0_ref, as1_ref, ad0_ref, ad1_ref,
                 tg_ref, ts_ref, td_ref):
    x1 = jnp.maximum(dis_ref[...] * agg_ref[...] + s1_ref[...] + b1_ref[...],
                     0.0)
    x1n = _ln(x1, lng_ref[...], lnb_ref[...])
    hh = x1n @ wgat_ref[...]
    als0 = jnp.sum(hh[:, :64] * as0_ref[...], axis=-1, keepdims=True)
    als1 = jnp.sum(hh[:, 64:] * as1_ref[...], axis=-1, keepdims=True)
    ald0 = jnp.sum(hh[:, :64] * ad0_ref[...], axis=-1, keepdims=True)
    ald1 = jnp.sum(hh[:, 64:] * ad1_ref[...], axis=-1, keepdims=True)
    z14 = jnp.zeros((hh.shape[0], 14), f32)
    tg_ref[...] = jnp.concatenate([hh, als0, als1, z14], axis=1)
    ts_ref[...] = jnp.concatenate([als0, als1, z14], axis=1)
    td_ref[...] = jnp.concatenate([ald0, ald1, z14], axis=1)


def _tc_gat_denom_fin(ts_ref, td_ref, s0_ref, s1_ref, dg_ref):
    e_self = ts_ref[...] + td_ref[...]
    p_self = jnp.exp(_lrelu(e_self))
    s_full = s0_ref[...] + s1_ref[...] + p_self
    inv = 1.0 / s_full
    z12 = jnp.zeros((e_self.shape[0], 12), f32)
    dg_ref[...] = jnp.concatenate(
        [td_ref[...][:, 0:2], inv[:, 0:2], z12], axis=1)


def _tc_gat_fin(o_ref, tg_ref, dg_ref, dis_ref, bg_ref, lng_ref, lnb_ref,
                w3_ref, a3_ref, s3_ref):
    tg = tg_ref[...]
    dg = dg_ref[...]
    hh = tg[:, 0:128]
    als = tg[:, 128:130]
    ald = dg[:, 0:2]
    inv = dg[:, 2:4]
    w = jnp.exp(_lrelu(als + ald)) * inv
    self_msg = hh[:, :64] * w[:, 0:1] + hh[:, 64:] * w[:, 1:2]
    x2 = jnp.maximum(0.5 * (o_ref[...] + self_msg) + bg_ref[...], 0.0)
    x2n = _ln(x2, lng_ref[...], lnb_ref[...])
    dis = dis_ref[...]
    hw3 = x2n @ w3_ref[...]
    a3_ref[...] = hw3 * dis
    s3_ref[...] = hw3 * (dis * dis)


def _tc_resid_fin(agg_ref, s3_ref, dis_ref, h_ref, b3_ref, lng_ref, lnb_ref,
                  xf_ref):
    x3 = jnp.maximum(dis_ref[...] * agg_ref[...] + s3_ref[...] + b3_ref[...],
                     0.0)
    xf_ref[...] = _ln(h_ref[...] + x3, lng_ref[...], lnb_ref[...])


def _tc_decode(z_ref, w1_ref, b1_ref, w2_ref, b2_ref, o_ref):
    z = jnp.maximum(z_ref[...] @ w1_ref[...] + b1_ref[...], 0.0)
    logits = z @ w2_ref[...] + b2_ref[...]
    o_ref[...] = 1.0 / (1.0 + jnp.exp(-logits))


def kernel(x, edge_index, edge_weight, src_idx, dst_idx,
           W_enc1, b_enc1, W_enc2, b_enc2,
           W_gcn1, b_gcn1, W_gat, a_src, a_dst, b_gat,
           W_gcn3, b_gcn3, ln_g, ln_b, Wv1, bv1, Wv2, bv2):
    row = edge_index[0].astype(i32)
    col = edge_index[1].astype(i32)
    ew = edge_weight.astype(f32)
    pad = EPAD - E
    r_pad = jnp.concatenate([row, jnp.zeros((pad,), i32)])
    c_pad = jnp.concatenate([col, jnp.full((pad,), N, i32)])
    w_pad = jnp.concatenate([ew, jnp.zeros((pad,), f32)])
    z16 = jnp.zeros((FACC, 16), f32)
    z64 = jnp.zeros((ACC, 64), f32)

    lng = ln_g.reshape(1, Hd)
    lnb = ln_b.reshape(1, Hd)

    h = pl.pallas_call(
        _tc_encode,
        grid=(GRID,),
        in_specs=[_row_spec(D), _full_spec(D, Hd), _full_spec(1, Hd),
                  _full_spec(Hd, Hd), _full_spec(1, Hd)],
        out_specs=_row_spec(Hd),
        out_shape=jax.ShapeDtypeStruct((N, Hd), f32),
    )(x, W_enc1.T, b_enc1.reshape(1, Hd), W_enc2.T, b_enc2.reshape(1, Hd))

    wrows = jnp.pad(w_pad[:, None], ((0, 0), (0, 15)))
    degp = _sc_degree(c_pad, wrows, z16)
    a1, s1, dis = pl.pallas_call(
        _tc_deg_scale,
        grid=(GRID,),
        in_specs=[_row_spec(Hd), _row_spec(16), _row_spec(16),
                  _full_spec(Hd, Hd)],
        out_specs=[_row_spec(Hd), _row_spec(Hd), _row_spec(1)],
        out_shape=(jax.ShapeDtypeStruct((N, Hd), f32),
                   jax.ShapeDtypeStruct((N, Hd), f32),
                   jax.ShapeDtypeStruct((N, 1), f32)),
    )(h, degp[0, :N], degp[1, :N], W_gcn1.T)

    agg1p = _sc_gcn_agg(a1, r_pad, c_pad, w_pad, z64)
    agg1 = jnp.concatenate([agg1p[0, :HALF], agg1p[1, :HALF]], axis=0)

    tg, t3s, t3d = pl.pallas_call(
        _tc_gat_prep,
        grid=(GRID,),
        in_specs=[_row_spec(Hd), _row_spec(Hd), _row_spec(1),
                  _full_spec(1, Hd), _full_spec(1, Hd), _full_spec(1, Hd),
                  _full_spec(Hd, 2 * Hd),
                  _full_spec(1, Hd), _full_spec(1, Hd),
                  _full_spec(1, Hd), _full_spec(1, Hd)],
        out_specs=[_row_spec(144), _row_spec(16), _row_spec(16)],
        out_shape=(jax.ShapeDtypeStruct((N, 144), f32),
                   jax.ShapeDtypeStruct((N, 16), f32),
                   jax.ShapeDtypeStruct((N, 16), f32)),
    )(agg1, s1, dis, b_gcn1.reshape(1, Hd), lng, lnb, W_gat.T,
      a_src[0].reshape(1, Hd), a_src[1].reshape(1, Hd),
      a_dst[0].reshape(1, Hd), a_dst[1].reshape(1, Hd))

    t3d_pad = jnp.pad(t3d, ((0, 8), (0, 0)))
    sden = _sc_gat_denom(t3s, t3d_pad, r_pad, c_pad, z16)

    dg = pl.pallas_call(
        _tc_gat_denom_fin,
        grid=(GRID,),
        in_specs=[_row_spec(16), _row_spec(16), _row_spec(16), _row_spec(16)],
        out_specs=_row_spec(16),
        out_shape=jax.ShapeDtypeStruct((N, 16), f32),
    )(t3s, t3d, sden[0, :N], sden[1, :N])

    dg_pad = jnp.pad(dg, ((0, 8), (0, 0)))
    oaggp = _sc_gat_agg(tg, dg_pad, r_pad, c_pad, z64)
    oagg = jnp.concatenate([oaggp[0, :HALF], oaggp[1, :HALF]], axis=0)

    a3, s3 = pl.pallas_call(
        _tc_gat_fin,
        grid=(GRID,),
        in_specs=[_row_spec(Hd), _row_spec(144), _row_spec(16), _row_spec(1),
                  _full_spec(1, Hd), _full_spec(1, Hd), _full_spec(1, Hd),
                  _full_spec(Hd, Hd)],
        out_specs=[_row_spec(Hd), _row_spec(Hd)],
        out_shape=(jax.ShapeDtypeStruct((N, Hd), f32),
                   jax.ShapeDtypeStruct((N, Hd), f32)),
    )(oagg, tg, dg, dis, b_gat.reshape(1, Hd), lng, lnb, W_gcn3.T)

    agg3p = _sc_gcn_agg(a3, r_pad, c_pad, w_pad, z64)
    agg3 = jnp.concatenate([agg3p[0, :HALF], agg3p[1, :HALF]], axis=0)

    xf = pl.pallas_call(
        _tc_resid_fin,
        grid=(GRID,),
        in_specs=[_row_spec(Hd), _row_spec(Hd), _row_spec(1), _row_spec(Hd),
                  _full_spec(1, Hd), _full_spec(1, Hd), _full_spec(1, Hd)],
        out_specs=_row_spec(Hd),
        out_shape=jax.ShapeDtypeStruct((N, Hd), f32),
    )(agg3, s3, dis, h, b_gcn3.reshape(1, Hd), lng, lnb)

    se, de = _sc_pair_gather(xf, src_idx.astype(i32), dst_idx.astype(i32))
    zcat = jnp.concatenate([se, de], axis=1)
    wv2p = jnp.pad(Wv2.T, ((0, 0), (0, 3)))
    bv2p = jnp.pad(bv2.reshape(1, 5), ((0, 0), (0, 3)))
    out8 = pl.pallas_call(
        _tc_decode,
        in_specs=[pl.BlockSpec((B, 2 * Hd), lambda: (0, 0)),
                  pl.BlockSpec((2 * Hd, Hd), lambda: (0, 0)),
                  pl.BlockSpec((1, Hd), lambda: (0, 0)),
                  pl.BlockSpec((Hd, 8), lambda: (0, 0)),
                  pl.BlockSpec((1, 8), lambda: (0, 0))],
        out_specs=pl.BlockSpec((B, 8), lambda: (0, 0)),
        out_shape=jax.ShapeDtypeStruct((B, 8), f32),
    )(zcat, Wv1.T, bv1.reshape(1, Hd), wv2p, bv2p)
    return out8[:, :5]


# speedup vs baseline: 1.0218x; 1.0218x over previous
import functools

import jax
import jax.numpy as jnp
from jax import lax
from jax.experimental import pallas as pl
from jax.experimental.pallas import tpu as pltpu
from jax.experimental.pallas import tpu_sc as plsc

N = 50000
E = 800000
D = 128
Hd = 64
B = 1024

EPAD = 802816
FACC = 50048
FSTRIPE = FACC // 16
HALF = 25000
ACC = 25088
ASTRIPE = ACC // 16
CH32 = EPAD // 32 // 128
CH16 = EPAD // 16 // 128
BLK16 = EPAD // 16 // 1024
BLK16G = EPAD // 16 // 512

_mesh = plsc.VectorSubcoreMesh(core_axis_name="c", subcore_axis_name="s")
f32 = jnp.float32
i32 = jnp.int32


def _lrelu(x):
    return jnp.maximum(x, 0.2 * x)


def _ln(h, g, b):
    m = jnp.mean(h, axis=-1, keepdims=True)
    v = jnp.mean((h - m) ** 2, axis=-1, keepdims=True)
    return (h - m) * lax.rsqrt(v + 1e-5) * g + b


@functools.partial(
    pl.kernel, mesh=_mesh,
    compiler_params=pltpu.CompilerParams(use_tc_tiling_on_sc=False),
    out_type=jax.ShapeDtypeStruct((2, FACC, 16), f32),
    scratch_types=[
        pltpu.VMEM((128,), i32),
        pltpu.VMEM((128, 16), f32),
        pltpu.VMEM_SHARED((FACC, 16), f32),
    ],
)
def _sc_degree(c_hbm, wrows_hbm, z16_hbm, out_hbm, cv, buf, acc):
    sc = lax.axis_index("c")
    s = lax.axis_index("s")
    wid = s * 2 + sc
    pltpu.sync_copy(z16_hbm.at[pl.ds(s * FSTRIPE, FSTRIPE)],
                    acc.at[pl.ds(s * FSTRIPE, FSTRIPE)])
    plsc.subcore_barrier()

    def chunk(g, _):
        off = (wid * CH32 + g) * 128
        pltpu.sync_copy(c_hbm.at[pl.ds(off, 128)], cv)
        pltpu.sync_copy(wrows_hbm.at[pl.ds(off, 128)], buf)
        pltpu.sync_copy(buf, acc.at[cv], add=True)
        return _

    lax.fori_loop(0, CH32, chunk, None)
    plsc.subcore_barrier()
    pltpu.sync_copy(acc.at[pl.ds(s * FSTRIPE, FSTRIPE)],
                    out_hbm.at[sc, pl.ds(s * FSTRIPE, FSTRIPE)])


@functools.partial(
    pl.kernel, mesh=_mesh,
    compiler_params=pltpu.CompilerParams(use_tc_tiling_on_sc=False),
    out_type=jax.ShapeDtypeStruct((2, ACC, 64), f32),
    scratch_types=[
        pltpu.VMEM((1024,), i32),
        pltpu.VMEM((1024,), i32),
        pltpu.VMEM((128,), i32),
        pltpu.VMEM((1040,), f32),
        pltpu.VMEM((128, 64), f32),
        pltpu.VMEM((128, 64), f32),
        pltpu.VMEM_SHARED((ACC, 64), f32),
        pltpu.SemaphoreType.DMA,
        pltpu.SemaphoreType.DMA,
    ],
)
def _sc_gcn_agg(table_hbm, r_hbm, c_hbm, w_hbm, z64_hbm, out_hbm,
                rv, cv, cl, wv, rows0, rows1, acc, sem0, sem1):
    sc = lax.axis_index("c")
    s = lax.axis_index("s")
    base = sc * HALF
    pltpu.sync_copy(z64_hbm.at[pl.ds(s * ASTRIPE, ASTRIPE)],
                    acc.at[pl.ds(s * ASTRIPE, ASTRIPE)])
    plsc.subcore_barrier()

    bufs = None

    def block(g, _):
        off = (s * BLK16 + g) * 1024
        pltpu.sync_copy(r_hbm.at[pl.ds(off, 1024)], rv)
        pltpu.sync_copy(c_hbm.at[pl.ds(off, 1024)], cv)
        pltpu.sync_copy(w_hbm.at[pl.ds(off, 1024)], wv.at[pl.ds(0, 1024)])
        bufs = (rows0, rows1)
        sems = (sem0, sem1)

        def fire(sub):
            return pltpu.async_copy(
                table_hbm.at[rv.at[pl.ds(sub * 128, 128)]],
                bufs[sub % 2], sems[sub % 2])

        hs = [None] * 8
        hs[0] = fire(0)
        for sub in range(8):
            if sub + 1 < 8:
                hs[sub + 1] = fire(sub + 1)
            hs[sub].wait()
            rows = bufs[sub % 2]
            sb = sub * 128
            for j in range(8):
                lv = cv[pl.ds(sb + j * 16, 16)] - base
                ok = (lv >= 0) & (lv < HALF)
                cl[pl.ds(j * 16, 16)] = jnp.where(ok, lv, HALF)

            def edge(e, __, rows=rows, sb=sb):
                w = wv[pl.ds(sb + e, 16)][0]
                for k in range(4):
                    sl = rows[e, pl.ds(k * 16, 16)]
                    rows[e, pl.ds(k * 16, 16)] = sl * w
                return __

            lax.fori_loop(0, 128, edge, None)
            pltpu.sync_copy(rows, acc.at[cl], add=True)
        return _

    lax.fori_loop(0, BLK16, block, None)
    plsc.subcore_barrier()
    pltpu.sync_copy(acc.at[pl.ds(s * ASTRIPE, ASTRIPE)],
                    out_hbm.at[sc, pl.ds(s * ASTRIPE, ASTRIPE)])


@functools.partial(
    pl.kernel, mesh=_mesh,
    compiler_params=pltpu.CompilerParams(use_tc_tiling_on_sc=False),
    out_type=jax.ShapeDtypeStruct((2, FACC, 16), f32),
    scratch_types=[
        pltpu.VMEM((128,), i32),
        pltpu.VMEM((128,), i32),
        pltpu.VMEM((128, 16), f32),
        pltpu.VMEM((128, 16), f32),
        pltpu.VMEM((128, 16), f32),
        pltpu.VMEM_SHARED((FACC, 16), f32),
        pltpu.SemaphoreType.DMA,
    ],
)
def _sc_gat_denom(ts_hbm, td_hbm, r_hbm, c_hbm, z16_hbm, out_hbm,
                  rv, cv, srows, drows, pbuf, acc, sem):
    sc = lax.axis_index("c")
    s = lax.axis_index("s")
    wid = s * 2 + sc
    pltpu.sync_copy(z16_hbm.at[pl.ds(s * FSTRIPE, FSTRIPE)],
                    acc.at[pl.ds(s * FSTRIPE, FSTRIPE)])
    plsc.subcore_barrier()

    def chunk(g, _):
        off = (wid * CH32 + g) * 128
        pltpu.sync_copy(r_hbm.at[pl.ds(off, 128)], rv)
        pltpu.sync_copy(c_hbm.at[pl.ds(off, 128)], cv)
        pltpu.async_copy(ts_hbm.at[rv], srows, sem).wait()
        pltpu.async_copy(td_hbm.at[cv], drows, sem).wait()

        def edge(e, __):
            ev = srows[e, pl.ds(0, 16)] + drows[e, pl.ds(0, 16)]
            pbuf[e, pl.ds(0, 16)] = jnp.exp(_lrelu(ev))
            return __

        lax.fori_loop(0, 128, edge, None)
        pltpu.sync_copy(pbuf, acc.at[cv], add=True)
        return _

    lax.fori_loop(0, CH32, chunk, None)
    plsc.subcore_barrier()
    pltpu.sync_copy(acc.at[pl.ds(s * FSTRIPE, FSTRIPE)],
                    out_hbm.at[sc, pl.ds(s * FSTRIPE, FSTRIPE)])


@functools.partial(
    pl.kernel, mesh=_mesh,
    compiler_params=pltpu.CompilerParams(use_tc_tiling_on_sc=False),
    out_type=jax.ShapeDtypeStruct((2, ACC, 64), f32),
    scratch_types=[
        pltpu.VMEM((512,), i32),
        pltpu.VMEM((512,), i32),
        pltpu.VMEM((64,), i32),
        pltpu.VMEM((64, 144), f32),
        pltpu.VMEM((64, 144), f32),
        pltpu.VMEM((64, 16), f32),
        pltpu.VMEM((64, 16), f32),
        pltpu.VMEM((64, 64), f32),
        pltpu.VMEM_SHARED((ACC, 64), f32),
        pltpu.SemaphoreType.DMA,
        pltpu.SemaphoreType.DMA,
        pltpu.SemaphoreType.DMA,
        pltpu.SemaphoreType.DMA,
    ],
)
def _sc_gat_agg(tg_hbm, dg_hbm, r_hbm, c_hbm, z64_hbm, out_hbm,
                rv, cv, cl, trows0, trows1, drows0, drows1, msg, acc,
                semt0, semt1, semd0, semd1):
    sc = lax.axis_index("c")
    s = lax.axis_index("s")
    base = sc * HALF
    pltpu.sync_copy(z64_hbm.at[pl.ds(s * ASTRIPE, ASTRIPE)],
                    acc.at[pl.ds(s * ASTRIPE, ASTRIPE)])
    plsc.subcore_barrier()

    def block(g, _):
        off = (s * BLK16G + g) * 512
        pltpu.sync_copy(r_hbm.at[pl.ds(off, 512)], rv)
        pltpu.sync_copy(c_hbm.at[pl.ds(off, 512)], cv)
        tbufs = (trows0, trows1)
        dbufs = (drows0, drows1)
        tsems = (semt0, semt1)
        dsems = (semd0, semd1)

        def fire(sub):
            ht = pltpu.async_copy(tg_hbm.at[rv.at[pl.ds(sub * 64, 64)]],
                                  tbufs[sub % 2], tsems[sub % 2])
            hd = pltpu.async_copy(dg_hbm.at[cv.at[pl.ds(sub * 64, 64)]],
                                  dbufs[sub % 2], dsems[sub % 2])
            return ht, hd

        hs = [None] * 8
        hs[0] = fire(0)
        for sub in range(8):
            if sub + 1 < 8:
                hs[sub + 1] = fire(sub + 1)
            hs[sub][0].wait()
            hs[sub][1].wait()
            trows = tbufs[sub % 2]
            drows = dbufs[sub % 2]
            sb = sub * 64
            for j in range(4):
                lv = cv[pl.ds(sb + j * 16, 16)] - base
                ok = (lv >= 0) & (lv < HALF)
                cl[pl.ds(j * 16, 16)] = jnp.where(ok, lv, HALF)

            def edge(e, __, trows=trows, drows=drows):
                dv = drows[e, pl.ds(0, 16)]
                ev = trows[e, pl.ds(128, 16)] + dv
                p = jnp.exp(_lrelu(ev))
                w0 = p[0] * dv[2]
                w1 = p[1] * dv[3]
                for k in range(4):
                    h0 = trows[e, pl.ds(k * 16, 16)]
                    h1 = trows[e, pl.ds(64 + k * 16, 16)]
                    msg[e, pl.ds(k * 16, 16)] = h0 * w0 + h1 * w1
                return __

            lax.fori_loop(0, 64, edge, None)
            pltpu.sync_copy(msg, acc.at[cl], add=True)
        return _

    lax.fori_loop(0, BLK16G, block, None)
    plsc.subcore_barrier()
    pltpu.sync_copy(acc.at[pl.ds(s * ASTRIPE, ASTRIPE)],
                    out_hbm.at[sc, pl.ds(s * ASTRIPE, ASTRIPE)])


@functools.partial(
    pl.kernel, mesh=_mesh,
    compiler_params=pltpu.CompilerParams(use_tc_tiling_on_sc=False),
    out_type=(jax.ShapeDtypeStruct((B, 64), f32),
              jax.ShapeDtypeStruct((B, 64), f32)),
    scratch_types=[
        pltpu.VMEM((32,), i32),
        pltpu.VMEM((32, 64), f32),
        pltpu.SemaphoreType.DMA,
    ],
)
def _sc_pair_gather(xf_hbm, src_hbm, dst_hbm, se_hbm, de_hbm, idx, rows, sem):
    sc = lax.axis_index("c")
    s = lax.axis_index("s")
    wid = s * 2 + sc
    bp = B // 32
    pltpu.sync_copy(src_hbm.at[pl.ds(wid * bp, bp)], idx)
    pltpu.async_copy(xf_hbm.at[idx], rows, sem).wait()
    pltpu.sync_copy(rows, se_hbm.at[pl.ds(wid * bp, bp)])
    pltpu.sync_copy(dst_hbm.at[pl.ds(wid * bp, bp)], idx)
    pltpu.async_copy(xf_hbm.at[idx], rows, sem).wait()
    pltpu.sync_copy(rows, de_hbm.at[pl.ds(wid * bp, bp)])


BM = 256
GRID = pl.cdiv(N, BM)


def _row_spec(w):
    return pl.BlockSpec((BM, w), lambda i: (i, 0))


def _full_spec(a, b):
    return pl.BlockSpec((a, b), lambda i: (0, 0))


def _tc_encode(x_ref, w1_ref, b1_ref, w2_ref, b2_ref, h_ref):
    t = jnp.maximum(x_ref[...] @ w1_ref[...] + b1_ref[...], 0.0)
    h_ref[...] = jnp.maximum(t @ w2_ref[...] + b2_ref[...], 0.0)


def _tc_deg_scale(h_ref, d0_ref, d1_ref, wg_ref, a1_ref, s1_ref, dis_ref):
    deg = 1.0 + d0_ref[...][:, 0:1] + d1_ref[...][:, 0:1]
    dis = lax.rsqrt(deg)
    hw = h_ref[...] @ wg_ref[...]
    a1_ref[...] = hw * dis
    s1_ref[...] = hw * (dis * dis)
    dis_ref[...] = dis


def _tc_gat_prep(agg_ref, s1_ref, dis_ref, b1_ref, lng_ref, lnb_ref,
                 wgat_ref, as0_ref, as1_ref, ad0_ref, ad1_ref,
                 tg_ref, ts_ref, td_ref):
    x1 = jnp.maximum(dis_ref[...] * agg_ref[...] + s1_ref[...] + b1_ref[...],
                     0.0)
    x1n = _ln(x1, lng_ref[...], lnb_ref[...])
    hh = x1n @ wgat_ref[...]
    als0 = jnp.sum(hh[:, :64] * as0_ref[...], axis=-1, keepdims=True)
    als1 = jnp.sum(hh[:, 64:] * as1_ref[...], axis=-1, keepdims=True)
    ald0 = jnp.sum(hh[:, :64] * ad0_ref[...], axis=-1, keepdims=True)
    ald1 = jnp.sum(hh[:, 64:] * ad1_ref[...], axis=-1, keepdims=True)
    z14 = jnp.zeros((hh.shape[0], 14), f32)
    tg_ref[...] = jnp.concatenate([hh, als0, als1, z14], axis=1)
    ts_ref[...] = jnp.concatenate([als0, als1, z14], axis=1)
    td_ref[...] = jnp.concatenate([ald0, ald1, z14], axis=1)


def _tc_gat_denom_fin(ts_ref, td_ref, s0_ref, s1_ref, dg_ref):
    e_self = ts_ref[...] + td_ref[...]
    p_self = jnp.exp(_lrelu(e_self))
    s_full = s0_ref[...] + s1_ref[...] + p_self
    inv = 1.0 / s_full
    z12 = jnp.zeros((e_self.shape[0], 12), f32)
    dg_ref[...] = jnp.concatenate(
        [td_ref[...][:, 0:2], inv[:, 0:2], z12], axis=1)


def _tc_gat_fin(o_ref, tg_ref, dg_ref, dis_ref, bg_ref, lng_ref, lnb_ref,
                w3_ref, a3_ref, s3_ref):
    tg = tg_ref[...]
    dg = dg_ref[...]
    hh = tg[:, 0:128]
    als = tg[:, 128:130]
    ald = dg[:, 0:2]
    inv = dg[:, 2:4]
    w = jnp.exp(_lrelu(als + ald)) * inv
    self_msg = hh[:, :64] * w[:, 0:1] + hh[:, 64:] * w[:, 1:2]
    x2 = jnp.maximum(0.5 * (o_ref[...] + self_msg) + bg_ref[...], 0.0)
    x2n = _ln(x2, lng_ref[...], lnb_ref[...])
    dis = dis_ref[...]
    hw3 = x2n @ w3_ref[...]
    a3_ref[...] = hw3 * dis
    s3_ref[...] = hw3 * (dis * dis)


def _tc_resid_fin(agg_ref, s3_ref, dis_ref, h_ref, b3_ref, lng_ref, lnb_ref,
                  xf_ref):
    x3 = jnp.maximum(dis_ref[...] * agg_ref[...] + s3_ref[...] + b3_ref[...],
                     0.0)
    xf_ref[...] = _ln(h_ref[...] + x3, lng_ref[...], lnb_ref[...])


def _tc_decode(z_ref, w1_ref, b1_ref, w2_ref, b2_ref, o_ref):
    z = jnp.maximum(z_ref[...] @ w1_ref[...] + b1_ref[...], 0.0)
    logits = z @ w2_ref[...] + b2_ref[...]
    o_ref[...] = 1.0 / (1.0 + jnp.exp(-logits))


def kernel(x, edge_index, edge_weight, src_idx, dst_idx,
           W_enc1, b_enc1, W_enc2, b_enc2,
           W_gcn1, b_gcn1, W_gat, a_src, a_dst, b_gat,
           W_gcn3, b_gcn3, ln_g, ln_b, Wv1, bv1, Wv2, bv2):
    row = edge_index[0].astype(i32)
    col = edge_index[1].astype(i32)
    ew = edge_weight.astype(f32)
    pad = EPAD - E
    r_pad = jnp.concatenate([row, jnp.zeros((pad,), i32)])
    c_pad = jnp.concatenate([col, jnp.full((pad,), N, i32)])
    w_pad = jnp.concatenate([ew, jnp.zeros((pad,), f32)])
    z16 = jnp.zeros((FACC, 16), f32)
    z64 = jnp.zeros((ACC, 64), f32)

    lng = ln_g.reshape(1, Hd)
    lnb = ln_b.reshape(1, Hd)

    h = pl.pallas_call(
        _tc_encode,
        grid=(GRID,),
        in_specs=[_row_spec(D), _full_spec(D, Hd), _full_spec(1, Hd),
                  _full_spec(Hd, Hd), _full_spec(1, Hd)],
        out_specs=_row_spec(Hd),
        out_shape=jax.ShapeDtypeStruct((N, Hd), f32),
    )(x, W_enc1.T, b_enc1.reshape(1, Hd), W_enc2.T, b_enc2.reshape(1, Hd))

    wrows = jnp.pad(w_pad[:, None], ((0, 0), (0, 15)))
    degp = _sc_degree(c_pad, wrows, z16)
    a1, s1, dis = pl.pallas_call(
        _tc_deg_scale,
        grid=(GRID,),
        in_specs=[_row_spec(Hd), _row_spec(16), _row_spec(16),
                  _full_spec(Hd, Hd)],
        out_specs=[_row_spec(Hd), _row_spec(Hd), _row_spec(1)],
        out_shape=(jax.ShapeDtypeStruct((N, Hd), f32),
                   jax.ShapeDtypeStruct((N, Hd), f32),
                   jax.ShapeDtypeStruct((N, 1), f32)),
    )(h, degp[0, :N], degp[1, :N], W_gcn1.T)

    agg1p = _sc_gcn_agg(a1, r_pad, c_pad, w_pad, z64)
    agg1 = jnp.concatenate([agg1p[0, :HALF], agg1p[1, :HALF]], axis=0)

    tg, t3s, t3d = pl.pallas_call(
        _tc_gat_prep,
        grid=(GRID,),
        in_specs=[_row_spec(Hd), _row_spec(Hd), _row_spec(1),
                  _full_spec(1, Hd), _full_spec(1, Hd), _full_spec(1, Hd),
                  _full_spec(Hd, 2 * Hd),
                  _full_spec(1, Hd), _full_spec(1, Hd),
                  _full_spec(1, Hd), _full_spec(1, Hd)],
        out_specs=[_row_spec(144), _row_spec(16), _row_spec(16)],
        out_shape=(jax.ShapeDtypeStruct((N, 144), f32),
                   jax.ShapeDtypeStruct((N, 16), f32),
                   jax.ShapeDtypeStruct((N, 16), f32)),
    )(agg1, s1, dis, b_gcn1.reshape(1, Hd), lng, lnb, W_gat.T,
      a_src[0].reshape(1, Hd), a_src[1].reshape(1, Hd),
      a_dst[0].reshape(1, Hd), a_dst[1].reshape(1, Hd))

    t3d_pad = jnp.pad(t3d, ((0, 8), (0, 0)))
    sden = _sc_gat_denom(t3s, t3d_pad, r_pad, c_pad, z16)

    dg = pl.pallas_call(
        _tc_gat_denom_fin,
        grid=(GRID,),
        in_specs=[_row_spec(16), _row_spec(16), _row_spec(16), _row_spec(16)],
        out_specs=_row_spec(16),
        out_shape=jax.ShapeDtypeStruct((N, 16), f32),
    )(t3s, t3d, sden[0, :N], sden[1, :N])

    dg_pad = jnp.pad(dg, ((0, 8), (0, 0)))
    oaggp = _sc_gat_agg(tg, dg_pad, r_pad, c_pad, z64)
    oagg = jnp.concatenate([oaggp[0, :HALF], oaggp[1, :HALF]], axis=0)

    a3, s3 = pl.pallas_call(
        _tc_gat_fin,
        grid=(GRID,),
        in_specs=[_row_spec(Hd), _row_spec(144), _row_spec(16), _row_spec(1),
                  _full_spec(1, Hd), _full_spec(1, Hd), _full_spec(1, Hd),
                  _full_spec(Hd, Hd)],
        out_specs=[_row_spec(Hd), _row_spec(Hd)],
        out_shape=(jax.ShapeDtypeStruct((N, Hd), f32),
                   jax.ShapeDtypeStruct((N, Hd), f32)),
    )(oagg, tg, dg, dis, b_gat.reshape(1, Hd), lng, lnb, W_gcn3.T)

    agg3p = _sc_gcn_agg(a3, r_pad, c_pad, w_pad, z64)
    agg3 = jnp.concatenate([agg3p[0, :HALF], agg3p[1, :HALF]], axis=0)

    xf = pl.pallas_call(
        _tc_resid_fin,
        grid=(GRID,),
        in_specs=[_row_spec(Hd), _row_spec(Hd), _row_spec(1), _row_spec(Hd),
                  _full_spec(1, Hd), _full_spec(1, Hd), _full_spec(1, Hd)],
        out_specs=_row_spec(Hd),
        out_shape=jax.ShapeDtypeStruct((N, Hd), f32),
    )(agg3, s3, dis, h, b_gcn3.reshape(1, Hd), lng, lnb)

    se, de = _sc_pair_gather(xf, src_idx.astype(i32), dst_idx.astype(i32))
    zcat = jnp.concatenate([se, de], axis=1)
    wv2p = jnp.pad(Wv2.T, ((0, 0), (0, 3)))
    bv2p = jnp.pad(bv2.reshape(1, 5), ((0, 0), (0, 3)))
    out8 = pl.pallas_call(
        _tc_decode,
        in_specs=[pl.BlockSpec((B, 2 * Hd), lambda: (0, 0)),
                  pl.BlockSpec((2 * Hd, Hd), lambda: (0, 0)),
                  pl.BlockSpec((1, Hd), lambda: (0, 0)),
                  pl.BlockSpec((Hd, 8), lambda: (0, 0)),
                  pl.BlockSpec((1, 8), lambda: (0, 0))],
        out_specs=pl.BlockSpec((B, 8), lambda: (0, 0)),
        out_shape=jax.ShapeDtypeStruct((B, 8), f32),
    )(zcat, Wv1.T, bv1.reshape(1, Hd), wv2p, bv2p)
    return out8[:, :5]
